# Initial kernel scaffold; baseline (speedup 1.0000x reference)
import jax
import jax.numpy as jnp
from jax.experimental import pallas as pl


def kernel(feats, scores, masks):
    raise NotImplementedError("write your pallas kernel here")


# speedup vs baseline: 2.6762x; 2.6762x over previous
import jax
import jax.numpy as jnp
from jax import lax
from jax.experimental import pallas as pl
from jax.experimental.pallas import tpu as pltpu


def _body(x_ref, out_ref, *, scale):
    i = pl.program_id(0)
    j = pl.program_id(1)

    @pl.when(jnp.logical_and(i == 0, j == 0))
    def _():
        out_ref[0, 0] = 0.0

    x = x_ref[...]
    s = jnp.sum(x * x, axis=2, keepdims=True)
    y = x * lax.rsqrt(jnp.maximum(s, 1e-24))
    d = jnp.abs(y[:-1] - y[1:])
    out_ref[0, 0] += jnp.sum(d) * scale


def kernel(feats, scores, masks):
    del scores, masks
    F, n, c, h, w = feats.shape
    hb = 16 if h % 16 == 0 else h
    n_h = h // hb
    scale = 1.0 / ((F - 1) * n * c * h * w)

    out = pl.pallas_call(
        lambda x_ref, out_ref: _body(x_ref, out_ref, scale=scale),
        grid=(n, n_h),
        in_specs=[
            pl.BlockSpec((F, 1, c, hb, w), lambda i, j: (0, i, 0, j, 0)),
        ],
        out_specs=pl.BlockSpec(
            (1, 1), lambda i, j: (0, 0), memory_space=pltpu.SMEM
        ),
        out_shape=jax.ShapeDtypeStruct((1, 1), jnp.float32),
    )(feats)
    return out[0, 0]


# speedup vs baseline: 3.5535x; 1.3278x over previous
import jax
import jax.numpy as jnp
from jax import lax
from jax.experimental import pallas as pl
from jax.experimental.pallas import tpu as pltpu


def _body(x_ref, out_ref, *, scale):
    i = pl.program_id(0)
    j = pl.program_id(1)

    @pl.when(jnp.logical_and(i == 0, j == 0))
    def _():
        out_ref[0, 0] = 0.0

    x = x_ref[...]
    s = jnp.sum(x * x, axis=2, keepdims=True)
    y = x * lax.rsqrt(jnp.maximum(s, 1e-24))
    d = jnp.abs(y[:-1] - y[1:])
    out_ref[0, 0] += jnp.sum(d) * scale


def kernel(feats, scores, masks):
    del scores, masks
    F, n, c, h, w = feats.shape
    hb = 32 if h % 32 == 0 else h
    n_h = h // hb
    scale = 1.0 / ((F - 1) * n * c * h * w)

    out = pl.pallas_call(
        lambda x_ref, out_ref: _body(x_ref, out_ref, scale=scale),
        grid=(n, n_h),
        in_specs=[
            pl.BlockSpec((F, 1, c, hb, w), lambda i, j: (0, i, 0, j, 0)),
        ],
        out_specs=pl.BlockSpec(
            (1, 1), lambda i, j: (0, 0), memory_space=pltpu.SMEM
        ),
        out_shape=jax.ShapeDtypeStruct((1, 1), jnp.float32),
    )(feats)
    return out[0, 0]


# speedup vs baseline: 3.6896x; 1.0383x over previous
import jax
import jax.numpy as jnp
from jax import lax
from jax.experimental import pallas as pl
from jax.experimental.pallas import tpu as pltpu


def _body(x_ref, out_ref, *, scale):
    i = pl.program_id(0)
    j = pl.program_id(1)

    @pl.when(jnp.logical_and(i == 0, j == 0))
    def _():
        out_ref[0, 0] = 0.0

    x = x_ref[...]
    s = jnp.sum(x * x, axis=2, keepdims=True)
    y = x * lax.rsqrt(jnp.maximum(s, 1e-24))
    d = jnp.abs(y[:-1] - y[1:])
    out_ref[0, 0] += jnp.sum(d) * scale


def kernel(feats, scores, masks):
    del scores, masks
    F, n, c, h, w = feats.shape
    hb = 64 if h % 64 == 0 else h
    n_h = h // hb
    scale = 1.0 / ((F - 1) * n * c * h * w)

    out = pl.pallas_call(
        lambda x_ref, out_ref: _body(x_ref, out_ref, scale=scale),
        grid=(n, n_h),
        in_specs=[
            pl.BlockSpec((F, 1, c, hb, w), lambda i, j: (0, i, 0, j, 0)),
        ],
        out_specs=pl.BlockSpec(
            (1, 1), lambda i, j: (0, 0), memory_space=pltpu.SMEM
        ),
        out_shape=jax.ShapeDtypeStruct((1, 1), jnp.float32),
    )(feats)
    return out[0, 0]
